# baseline (speedup 1.0000x reference)
import functools

import jax
import jax.numpy as jnp
from jax import lax
from jax.experimental import pallas as pl
from jax.experimental.pallas import tpu as pltpu
from jax.experimental.pallas import tpu_sc as plsc

EMB = 128
N_X = 10000
N_EA = 320000
NC, NS = 2, 16
NW = NC * NS

X_PAD = 10240
N_ROWS = 2 * X_PAD + N_EA
N_ROWS_PAD = 344064
PER_W = N_ROWS_PAD // NW
CH = 224
NCH = PER_W // CH
N_TAB = 304


def _assemble(table_v, idx_v, rows_v, c, iota16):
    iota128 = iota16 * EMB

    def group(g, carry):
        idxv = idx_v[pl.ds(c * CH + g * 16, 16)]
        base = idxv * EMB
        dst0 = iota128 + g * (16 * EMB)
        for j in range(EMB):
            val = plsc.load_gather(table_v, [base + j])
            plsc.store_scatter(rows_v, [dst0 + j], val)
        return carry

    lax.fori_loop(0, CH // 16, group, 0)


def _start_out(rows_v, out, wbase, c, sem):
    pltpu.async_copy(rows_v, out.at[pl.ds((wbase + c * CH) * EMB, CH * EMB)],
                     sem)


def _wait_out(rows_v, out, wbase, sem):
    pltpu.make_async_copy(rows_v, out.at[pl.ds(wbase * EMB, CH * EMB)],
                          sem).wait()


def _body(idx_hbm, table_hbm, out_hbm, idx_v, table_v, rows0, rows1, o0, o1):
    wid = lax.axis_index("s") * NC + lax.axis_index("c")
    wbase = wid * PER_W
    pltpu.sync_copy(table_hbm, table_v)
    pltpu.sync_copy(idx_hbm.at[pl.ds(wbase, PER_W)], idx_v)
    iota16 = lax.iota(jnp.int32, 16)

    out2d = out_hbm

    def step(k, carry):
        c = 2 * k

        @pl.when(k > 0)
        def _():
            _wait_out(rows0, out2d, wbase, o0)

        _assemble(table_v, idx_v, rows0, c, iota16)
        _start_out(rows0, out2d, wbase, c, o0)

        @pl.when(k > 0)
        def _():
            _wait_out(rows1, out2d, wbase, o1)

        _assemble(table_v, idx_v, rows1, c + 1, iota16)
        _start_out(rows1, out2d, wbase, c + 1, o1)
        return carry

    lax.fori_loop(0, NCH // 2, step, 0)
    _wait_out(rows0, out2d, wbase, o0)
    _wait_out(rows1, out2d, wbase, o1)


_gather_all = functools.partial(
    pl.kernel,
    out_type=jax.ShapeDtypeStruct((N_ROWS_PAD * EMB,), jnp.float32),
    scratch_types=[
        pltpu.VMEM((PER_W,), jnp.int32),
        pltpu.VMEM((N_TAB * EMB,), jnp.float32),
        pltpu.VMEM((CH * EMB,), jnp.float32),
        pltpu.VMEM((CH * EMB,), jnp.float32),
        pltpu.SemaphoreType.DMA,
        pltpu.SemaphoreType.DMA,
    ],
    mesh=plsc.VectorSubcoreMesh(core_axis_name="c", subcore_axis_name="s"),
    compiler_params=pltpu.CompilerParams(needs_layout_passes=False),
)(_body)


def kernel(x, edge_attr, tuplefeat, x_table, ea_table, tf_table):
    pair_table = jnp.concatenate(
        [jnp.repeat(tf_table, 16, axis=0), jnp.tile(tf_table, (16, 1))], axis=1)
    table = jnp.concatenate([x_table, pair_table, ea_table], axis=0)

    tf = tuplefeat.astype(jnp.int32)
    tf_i = tf[:, 0] * 16 + tf[:, 1] + 32
    pad = jnp.zeros((X_PAD - N_X,), jnp.int32)
    idx = jnp.concatenate([
        x.reshape(-1).astype(jnp.int32), pad,
        tf_i, pad,
        edge_attr.astype(jnp.int32) + 32 + 256,
        jnp.zeros((N_ROWS_PAD - N_ROWS,), jnp.int32),
    ])

    out = _gather_all(idx, table.reshape(-1))
    out = out.reshape(N_ROWS_PAD, EMB)
    return (out[:N_X],
            out[2 * X_PAD:2 * X_PAD + N_EA],
            out[X_PAD:X_PAD + N_X])


# speedup vs baseline: 1.4351x; 1.4351x over previous
import functools

import jax
import jax.numpy as jnp
from jax import lax
from jax.experimental import pallas as pl
from jax.experimental.pallas import tpu as pltpu
from jax.experimental.pallas import tpu_sc as plsc

EMB = 128
N_X = 10000
N_EA = 320000
NC, NS = 2, 16
NW = NC * NS

X_PAD = 10240
N_ROWS = 2 * X_PAD + N_EA
N_ROWS_PAD = 344064
PER_W = N_ROWS_PAD // NW
CH = 112
NCH = PER_W // CH
NBUF = 6
N_TAB = 304


def _fire_gather(table, idx_v, c, rows_v, sem):
    pltpu.async_copy(table.at[idx_v.at[pl.ds(c * CH, CH)]], rows_v, sem)


def _wait_gather(table, rows_v, sem):
    pltpu.make_async_copy(table.at[pl.ds(0, CH)], rows_v, sem).wait()


def _fire_out(rows_v, out, wbase, c, sem):
    pltpu.async_copy(rows_v, out.at[pl.ds(wbase + c * CH, CH)], sem)


def _wait_out(rows_v, out, wbase, sem):
    pltpu.make_async_copy(rows_v, out.at[pl.ds(wbase, CH)], sem).wait()


def _body(idx_hbm, table_hbm, out_hbm, idx_v, *bufs_and_sems):
    rows = bufs_and_sems[:NBUF]
    gsem = bufs_and_sems[NBUF:2 * NBUF]
    osem = bufs_and_sems[2 * NBUF:3 * NBUF]
    wid = lax.axis_index("s") * NC + lax.axis_index("c")
    wbase = wid * PER_W
    pltpu.sync_copy(idx_hbm.at[pl.ds(wbase, PER_W)], idx_v)

    for b in range(NBUF):
        _fire_gather(table_hbm, idx_v, b, rows[b], gsem[b])

    def step(k, carry):
        c0 = k * NBUF
        for b in range(NBUF):
            _wait_gather(table_hbm, rows[b], gsem[b])
            _fire_out(rows[b], out_hbm, wbase, c0 + b, osem[b])
        for b in range(NBUF):
            _wait_out(rows[b], out_hbm, wbase, osem[b])
            _fire_gather(table_hbm, idx_v, c0 + NBUF + b, rows[b], gsem[b])
        return carry

    lax.fori_loop(0, NCH // NBUF - 1, step, 0)

    c0 = NCH - NBUF
    for b in range(NBUF):
        _wait_gather(table_hbm, rows[b], gsem[b])
        _fire_out(rows[b], out_hbm, wbase, c0 + b, osem[b])
    for b in range(NBUF):
        _wait_out(rows[b], out_hbm, wbase, osem[b])


_gather_all = functools.partial(
    pl.kernel,
    out_type=jax.ShapeDtypeStruct((N_ROWS_PAD, EMB), jnp.float32),
    scratch_types=(
        [pltpu.VMEM((PER_W,), jnp.int32)]
        + [pltpu.VMEM((CH, EMB), jnp.float32)] * NBUF
        + [pltpu.SemaphoreType.DMA] * (2 * NBUF)
    ),
    mesh=plsc.VectorSubcoreMesh(core_axis_name="c", subcore_axis_name="s"),
)(_body)


def kernel(x, edge_attr, tuplefeat, x_table, ea_table, tf_table):
    pair_table = jnp.concatenate(
        [jnp.repeat(tf_table, 16, axis=0), jnp.tile(tf_table, (16, 1))], axis=1)
    table = jnp.concatenate([x_table, pair_table, ea_table], axis=0)

    tf = tuplefeat.astype(jnp.int32)
    tf_i = tf[:, 0] * 16 + tf[:, 1] + 32
    pad = jnp.zeros((X_PAD - N_X,), jnp.int32)
    idx = jnp.concatenate([
        x.reshape(-1).astype(jnp.int32), pad,
        tf_i, pad,
        edge_attr.astype(jnp.int32) + 32 + 256,
        jnp.zeros((N_ROWS_PAD - N_ROWS,), jnp.int32),
    ])

    out = _gather_all(idx, table)
    return (out[:N_X],
            out[2 * X_PAD:2 * X_PAD + N_EA],
            out[X_PAD:X_PAD + N_X])


# speedup vs baseline: 1.6167x; 1.1266x over previous
import functools

import jax
import jax.numpy as jnp
from jax import lax
from jax.experimental import pallas as pl
from jax.experimental.pallas import tpu as pltpu
from jax.experimental.pallas import tpu_sc as plsc

EMB = 128
N_X = 10000
N_EA = 320000
NC, NS = 2, 16
NW = NC * NS

X_PAD = 10240
N_ROWS = 2 * X_PAD + N_EA
N_ROWS_PAD = 344064
PER_W = N_ROWS_PAD // NW
CH = 224
NCH = PER_W // CH
N_TAB = 304
JB = 16


def _assemble(table_v, idx_v, rows_v, c, iota16):
    iota128 = iota16 * EMB

    @plsc.parallel_loop(0, CH // 16, unroll=2)
    def group(g):
        idxv = idx_v[pl.ds(c * CH + g * 16, 16)]
        base = idxv * EMB
        dst0 = iota128 + g * (16 * EMB)
        for jb in range(0, EMB, JB):
            vals = [plsc.load_gather(table_v, [base + (jb + j)])
                    for j in range(JB)]
            for j in range(JB):
                plsc.store_scatter(rows_v, [dst0 + (jb + j)], vals[j])


def _start_out(rows_v, out, wbase, c, sem):
    pltpu.async_copy(rows_v, out.at[pl.ds((wbase + c * CH) * EMB, CH * EMB)],
                     sem)


def _wait_out(rows_v, out, wbase, sem):
    pltpu.make_async_copy(rows_v, out.at[pl.ds(wbase * EMB, CH * EMB)],
                          sem).wait()


def _body(idx_hbm, table_hbm, out_hbm, idx_v, table_v, rows0, rows1, o0, o1):
    wid = lax.axis_index("s") * NC + lax.axis_index("c")
    wbase = wid * PER_W
    pltpu.sync_copy(table_hbm, table_v)
    pltpu.sync_copy(idx_hbm.at[pl.ds(wbase, PER_W)], idx_v)
    iota16 = lax.iota(jnp.int32, 16)

    def step(k, carry):
        c = 2 * k

        @pl.when(k > 0)
        def _():
            _wait_out(rows0, out_hbm, wbase, o0)

        _assemble(table_v, idx_v, rows0, c, iota16)
        _start_out(rows0, out_hbm, wbase, c, o0)

        @pl.when(k > 0)
        def _():
            _wait_out(rows1, out_hbm, wbase, o1)

        _assemble(table_v, idx_v, rows1, c + 1, iota16)
        _start_out(rows1, out_hbm, wbase, c + 1, o1)
        return carry

    lax.fori_loop(0, NCH // 2, step, 0)
    _wait_out(rows0, out_hbm, wbase, o0)
    _wait_out(rows1, out_hbm, wbase, o1)


_gather_all = functools.partial(
    pl.kernel,
    out_type=jax.ShapeDtypeStruct((N_ROWS_PAD * EMB,), jnp.float32),
    scratch_types=[
        pltpu.VMEM((PER_W,), jnp.int32),
        pltpu.VMEM((N_TAB * EMB,), jnp.float32),
        pltpu.VMEM((CH * EMB,), jnp.float32),
        pltpu.VMEM((CH * EMB,), jnp.float32),
        pltpu.SemaphoreType.DMA,
        pltpu.SemaphoreType.DMA,
    ],
    mesh=plsc.VectorSubcoreMesh(core_axis_name="c", subcore_axis_name="s"),
    compiler_params=pltpu.CompilerParams(needs_layout_passes=False),
)(_body)


def kernel(x, edge_attr, tuplefeat, x_table, ea_table, tf_table):
    pair_table = jnp.concatenate(
        [jnp.repeat(tf_table, 16, axis=0), jnp.tile(tf_table, (16, 1))], axis=1)
    table = jnp.concatenate([x_table, pair_table, ea_table], axis=0)

    tf = tuplefeat.astype(jnp.int32)
    tf_i = tf[:, 0] * 16 + tf[:, 1] + 32
    pad = jnp.zeros((X_PAD - N_X,), jnp.int32)
    idx = jnp.concatenate([
        x.reshape(-1).astype(jnp.int32), pad,
        tf_i, pad,
        edge_attr.astype(jnp.int32) + 32 + 256,
        jnp.zeros((N_ROWS_PAD - N_ROWS,), jnp.int32),
    ])

    out = _gather_all(idx, table.reshape(-1))
    out = out.reshape(N_ROWS_PAD, EMB)
    return (out[:N_X],
            out[2 * X_PAD:2 * X_PAD + N_EA],
            out[X_PAD:X_PAD + N_X])


# speedup vs baseline: 8.1607x; 5.0477x over previous
import functools

import jax
import jax.numpy as jnp
from jax import lax
from jax.experimental import pallas as pl
from jax.experimental.pallas import tpu as pltpu
from jax.experimental.pallas import tpu_sc as plsc

EMB = 128
N_X = 10000
N_EA = 320000
NC, NS = 2, 16
NW = NC * NS

X_PAD = 10240
N_ROWS = 2 * X_PAD + N_EA
N_ROWS_PAD = 344064
PER_W = N_ROWS_PAD // NW
CH = 112
NCH = PER_W // CH
NBUF = 6
N_TAB = 304


def _fire_gather(table, idx_v, c, rows_v, sem):
    pltpu.async_copy(table.at[idx_v.at[pl.ds(c * CH, CH)]], rows_v, sem)


def _wait_gather(table, rows_v, sem):
    pltpu.make_async_copy(table.at[pl.ds(0, CH)], rows_v, sem).wait()


def _fire_out(rows_v, out, wbase, c, sem):
    pltpu.async_copy(rows_v, out.at[pl.ds(wbase + c * CH, CH)], sem)


def _wait_out(rows_v, out, wbase, sem):
    pltpu.make_async_copy(rows_v, out.at[pl.ds(wbase, CH)], sem).wait()


def _body(idx_hbm, table_hbm, out_hbm, idx_v, table_s, *bufs_and_sems):
    rows = bufs_and_sems[:NBUF]
    gsem = bufs_and_sems[NBUF:2 * NBUF]
    osem = bufs_and_sems[2 * NBUF:3 * NBUF]
    sid = lax.axis_index("s")
    wid = sid * NC + lax.axis_index("c")
    wbase = wid * PER_W

    @pl.when(sid == 0)
    def _():
        pltpu.sync_copy(table_hbm, table_s)

    pltpu.sync_copy(idx_hbm.at[pl.ds(wbase, PER_W)], idx_v)
    plsc.subcore_barrier()

    for b in range(NBUF):
        _fire_gather(table_s, idx_v, b, rows[b], gsem[b])

    def step(k, carry):
        c0 = k * NBUF
        for b in range(NBUF):
            _wait_gather(table_s, rows[b], gsem[b])
            _fire_out(rows[b], out_hbm, wbase, c0 + b, osem[b])
        for b in range(NBUF):
            _wait_out(rows[b], out_hbm, wbase, osem[b])
            _fire_gather(table_s, idx_v, c0 + NBUF + b, rows[b], gsem[b])
        return carry

    lax.fori_loop(0, NCH // NBUF - 1, step, 0)

    c0 = NCH - NBUF
    for b in range(NBUF):
        _wait_gather(table_s, rows[b], gsem[b])
        _fire_out(rows[b], out_hbm, wbase, c0 + b, osem[b])
    for b in range(NBUF):
        _wait_out(rows[b], out_hbm, wbase, osem[b])


_gather_all = functools.partial(
    pl.kernel,
    out_type=jax.ShapeDtypeStruct((N_ROWS_PAD, EMB), jnp.float32),
    scratch_types=(
        [pltpu.VMEM((PER_W,), jnp.int32),
         pltpu.VMEM_SHARED((N_TAB, EMB), jnp.float32)]
        + [pltpu.VMEM((CH, EMB), jnp.float32)] * NBUF
        + [pltpu.SemaphoreType.DMA] * (2 * NBUF)
    ),
    mesh=plsc.VectorSubcoreMesh(core_axis_name="c", subcore_axis_name="s"),
    compiler_params=pltpu.CompilerParams(needs_layout_passes=False),
)(_body)


def kernel(x, edge_attr, tuplefeat, x_table, ea_table, tf_table):
    pair_table = jnp.concatenate(
        [jnp.repeat(tf_table, 16, axis=0), jnp.tile(tf_table, (16, 1))], axis=1)
    table = jnp.concatenate([x_table, pair_table, ea_table], axis=0)

    tf = tuplefeat.astype(jnp.int32)
    tf_i = tf[:, 0] * 16 + tf[:, 1] + 32
    pad = jnp.zeros((X_PAD - N_X,), jnp.int32)
    idx = jnp.concatenate([
        x.reshape(-1).astype(jnp.int32), pad,
        tf_i, pad,
        edge_attr.astype(jnp.int32) + 32 + 256,
        jnp.zeros((N_ROWS_PAD - N_ROWS,), jnp.int32),
    ])

    out = _gather_all(idx, table)
    return (out[:N_X],
            out[2 * X_PAD:2 * X_PAD + N_EA],
            out[X_PAD:X_PAD + N_X])
